# baseline (speedup 1.0000x reference)
import functools

import jax
import jax.numpy as jnp
from jax import lax
from jax.experimental import pallas as pl
from jax.experimental.pallas import tpu as pltpu, tpu_sc as plsc

T = 16
NC = 2
K = 1024


def _radix_body(meta, e0_hbm, e1_hbm, ev_hbm, oc0_hbm, oc1_hbm, oval_hbm,
                e0_v, e1_v, val_v, keys_v, dst2_v, hista_v, histb_v, hist_v,
                offa_v, offb_v, grid_v,
                grid_sh, bufa_k, bufa_v, bufb_k, bufb_v, sem):
    B, N, M = meta
    E = M // T
    H = E // 2
    EP = ((E + 127) // 128) * 128
    VPH = H // 16
    BPC = B // NC
    c = lax.axis_index("c")
    t = lax.axis_index("s")
    iota = lax.iota(jnp.int32, 16)
    ones = jnp.ones((16,), jnp.int32)
    zeros = jnp.zeros((16,), jnp.int32)

    for z in range((EP - E) // 16):
        row, col = (E + z * 16) // 128, (E + z * 16) % 128
        dst2_v[row, pl.ds(col, 16)] = M + z * 16 + iota

    def dst_store(flat, vec):
        row = lax.shift_right_logical(flat, 7)
        col = lax.bitwise_and(flat, 127)
        dst2_v[row, pl.ds(col, 16)] = vec

    def zero_hists():
        @plsc.parallel_loop(0, K // 16, unroll=4)
        def _(z):
            hista_v[pl.ds(z * 16, 16)] = zeros
            histb_v[pl.ds(z * 16, 16)] = zeros

    def merge_hists():
        @plsc.parallel_loop(0, K // 16, unroll=4)
        def _(z):
            hist_v[pl.ds(z * 16, 16)] = (
                hista_v[pl.ds(z * 16, 16)] + histb_v[pl.ds(z * 16, 16)])

    def scan_offsets():
        pltpu.sync_copy(grid_sh, grid_v)

        @plsc.parallel_loop(0, K // 16, unroll=2, carry=jnp.int32(0))
        def _(z, carry):
            col = zeros
            pre = zeros
            for tt in range(T):
                h = grid_v[pl.ds(tt * K + z * 16, 16)]
                pre = pre + h * (t > tt).astype(jnp.int32)
                col = col + h
            incl = plsc.cumsum(col)
            offa = incl - col + pre + carry
            offa_v[pl.ds(z * 16, 16)] = offa
            offb_v[pl.ds(z * 16, 16)] = offa + hista_v[pl.ds(z * 16, 16)]
            return carry + jnp.sum(col)

    def rank_pass(shift):
        def digit(kk):
            if shift:
                return lax.shift_right_logical(kk, shift)
            return lax.bitwise_and(kk, K - 1)

        def rb(i, carry):
            ka = keys_v[pl.ds(i * 16, 16)]
            kb = keys_v[pl.ds(H + i * 16, 16)]
            da = digit(ka)
            db = digit(kb)
            ca, _ = plsc.scan_count(da)
            cb, _ = plsc.scan_count(db)
            basea = plsc.load_gather(offa_v, [da])
            baseb = plsc.load_gather(offb_v, [db])
            dst_store(i * 16, basea + ca - 1)
            dst_store(H + i * 16, baseb + cb - 1)
            plsc.addupdate_scatter(offa_v, [da], ones)
            plsc.addupdate_scatter(offb_v, [db], ones)
            return carry
        lax.fori_loop(0, VPH, rb, 0)

    def scatter_to(buf_k, buf_v):
        copies = []
        for row in range(EP // 128):
            copies.append(pltpu.async_copy(
                keys_v.at[pl.ds(row * 128, 128)],
                buf_k.at[dst2_v.at[row]], sem))
            copies.append(pltpu.async_copy(
                val_v.at[pl.ds(row * 128, 128)],
                buf_v.at[dst2_v.at[row]], sem))
        for cp in copies:
            cp.wait()

    def batch(j, carry):
        b = c * BPC + j
        base = b * M + t * E
        with jax.named_scope("ph_stage"):
            pltpu.sync_copy(e0_hbm.at[pl.ds(base, E)], e0_v)
            pltpu.sync_copy(e1_hbm.at[pl.ds(base, E)], e1_v)
            pltpu.sync_copy(ev_hbm.at[pl.ds(base, E)], val_v.at[pl.ds(0, E)])

        zero_hists()

        @plsc.parallel_loop(0, VPH, unroll=2)
        def _(i):
            e0a = e0_v[pl.ds(i * 16, 16)]
            e1a = e1_v[pl.ds(i * 16, 16)]
            e0b = e0_v[pl.ds(H + i * 16, 16)]
            e1b = e1_v[pl.ds(H + i * 16, 16)]
            keys_v[pl.ds(i * 16, 16)] = e0a * K + e1a
            keys_v[pl.ds(H + i * 16, 16)] = e0b * K + e1b
            plsc.addupdate_scatter(hista_v, [e1a], ones)
            plsc.addupdate_scatter(histb_v, [e1b], ones)
        with jax.named_scope("ph_merge1"):
            merge_hists()
            pltpu.sync_copy(hist_v, grid_sh.at[pl.ds(t * K, K)])
        with jax.named_scope("ph_bar1"):
            plsc.subcore_barrier()
        with jax.named_scope("ph_scan1"):
            scan_offsets()
        with jax.named_scope("ph_rank1"):
            rank_pass(0)
        with jax.named_scope("ph_scat1"):
            scatter_to(bufa_k, bufa_v)
        with jax.named_scope("ph_bar2"):
            plsc.subcore_barrier()

        with jax.named_scope("ph_stage2"):
            pltpu.sync_copy(bufa_k.at[pl.ds(t * E, E)], keys_v.at[pl.ds(0, E)])
            pltpu.sync_copy(bufa_v.at[pl.ds(t * E, E)], val_v.at[pl.ds(0, E)])
        zero_hists()

        @plsc.parallel_loop(0, VPH, unroll=2)
        def _(i):
            ka = keys_v[pl.ds(i * 16, 16)]
            kb = keys_v[pl.ds(H + i * 16, 16)]
            plsc.addupdate_scatter(
                hista_v, [lax.shift_right_logical(ka, 10)], ones)
            plsc.addupdate_scatter(
                histb_v, [lax.shift_right_logical(kb, 10)], ones)
        with jax.named_scope("ph_merge2"):
            merge_hists()
            pltpu.sync_copy(hist_v, grid_sh.at[pl.ds(t * K, K)])
        with jax.named_scope("ph_bar3"):
            plsc.subcore_barrier()
        with jax.named_scope("ph_scan2"):
            scan_offsets()
        with jax.named_scope("ph_rank2"):
            rank_pass(10)
        with jax.named_scope("ph_scat2"):
            scatter_to(bufb_k, bufb_v)
        with jax.named_scope("ph_bar4"):
            plsc.subcore_barrier()

        pltpu.sync_copy(bufb_k.at[pl.ds(t * E, E)], keys_v.at[pl.ds(0, E)])
        pltpu.sync_copy(bufb_v.at[pl.ds(t * E, E)], val_v.at[pl.ds(0, E)])
        base_node = b * N

        @plsc.parallel_loop(0, 2 * VPH, unroll=4)
        def _(i):
            kk = keys_v[pl.ds(i * 16, 16)]
            e0_v[pl.ds(i * 16, 16)] = \
                lax.shift_right_logical(kk, 10) + base_node
            e1_v[pl.ds(i * 16, 16)] = \
                lax.bitwise_and(kk, K - 1) + base_node
        pltpu.sync_copy(e0_v, oc0_hbm.at[pl.ds(base, E)])
        pltpu.sync_copy(e1_v, oc1_hbm.at[pl.ds(base, E)])
        pltpu.sync_copy(val_v.at[pl.ds(0, E)], oval_hbm.at[pl.ds(base, E)])
        return carry

    lax.fori_loop(0, BPC, batch, 0)


def kernel(nodes, edges, edge_index):
    b, n, f = nodes.shape
    m = edge_index.shape[1]
    e = m // T
    ep = ((e + 127) // 128) * 128
    e0f = edge_index[:, :, 0].reshape(b * m)
    e1f = edge_index[:, :, 1].reshape(b * m)
    ev = edges.reshape(b * m)
    mesh = plsc.VectorSubcoreMesh(core_axis_name="c", subcore_axis_name="s")
    fn = pl.kernel(
        functools.partial(_radix_body, (b, n, m)),
        out_type=(jax.ShapeDtypeStruct((b * m,), jnp.int32),
                  jax.ShapeDtypeStruct((b * m,), jnp.int32),
                  jax.ShapeDtypeStruct((b * m,), jnp.float32)),
        mesh=mesh,
        compiler_params=pltpu.CompilerParams(needs_layout_passes=False),
        scratch_types=[
            pltpu.VMEM((e,), jnp.int32),
            pltpu.VMEM((e,), jnp.int32),
            pltpu.VMEM((ep,), jnp.float32),
            pltpu.VMEM((ep,), jnp.int32),
            pltpu.VMEM((ep // 128, 128), jnp.int32),
            pltpu.VMEM((K,), jnp.int32),
            pltpu.VMEM((K,), jnp.int32),
            pltpu.VMEM((K,), jnp.int32),
            pltpu.VMEM((K,), jnp.int32),
            pltpu.VMEM((K,), jnp.int32),
            pltpu.VMEM((T * K,), jnp.int32),
            pltpu.VMEM_SHARED((T * K,), jnp.int32),
            pltpu.VMEM_SHARED((m + 128,), jnp.int32),
            pltpu.VMEM_SHARED((m + 128,), jnp.float32),
            pltpu.VMEM_SHARED((m + 128,), jnp.int32),
            pltpu.VMEM_SHARED((m + 128,), jnp.float32),
            pltpu.SemaphoreType.DMA,
        ],
    )
    oc0, oc1, oval = fn(e0f, e1f, ev)
    indexlist = jnp.stack([oc0, oc1], axis=1).astype(jnp.int64)
    dense_shape = jnp.array([b * n, b * n], dtype=jnp.int64)
    return indexlist, oval, dense_shape


# speedup vs baseline: 1.1849x; 1.1849x over previous
import functools

import jax
import jax.numpy as jnp
from jax import lax
from jax.experimental import pallas as pl
from jax.experimental.pallas import tpu as pltpu, tpu_sc as plsc

T = 16
NC = 2
K = 1024


def _radix_body(meta, e0_hbm, e1_hbm, ev_hbm, oc0_hbm, oc1_hbm, oval_hbm,
                e0_v, e1_v, val_v, keys_v, dst2_v, hista_v, histb_v, hist_v,
                offa_v, offb_v, grid_v, oc0s_v, oc1s_v, ovs_v,
                grid_sh, bufa_k, bufa_v, bufb_k, bufb_v,
                sem, sem_in, sem_out):
    B, N, M = meta
    E = M // T
    H = E // 2
    EP = ((E + 127) // 128) * 128
    VPH = H // 16
    BPC = B // NC
    c = lax.axis_index("c")
    t = lax.axis_index("s")
    iota = lax.iota(jnp.int32, 16)
    ones = jnp.ones((16,), jnp.int32)
    zeros = jnp.zeros((16,), jnp.int32)

    for z in range((EP - E) // 16):
        row, col = (E + z * 16) // 128, (E + z * 16) % 128
        dst2_v[row, pl.ds(col, 16)] = M + z * 16 + iota

    def dst_store(flat, vec):
        row = lax.shift_right_logical(flat, 7)
        col = lax.bitwise_and(flat, 127)
        dst2_v[row, pl.ds(col, 16)] = vec

    def zero_hists():
        @plsc.parallel_loop(0, K // 16, unroll=4)
        def _(z):
            hista_v[pl.ds(z * 16, 16)] = zeros
            histb_v[pl.ds(z * 16, 16)] = zeros

    def merge_hists():
        @plsc.parallel_loop(0, K // 16, unroll=4)
        def _(z):
            hist_v[pl.ds(z * 16, 16)] = (
                hista_v[pl.ds(z * 16, 16)] + histb_v[pl.ds(z * 16, 16)])

    def scan_offsets():
        pltpu.sync_copy(grid_sh, grid_v)

        @plsc.parallel_loop(0, K // 16, unroll=2, carry=jnp.int32(0))
        def _(z, carry):
            col = zeros
            pre = zeros
            for tt in range(T):
                h = grid_v[pl.ds(tt * K + z * 16, 16)]
                pre = pre + h * (t > tt).astype(jnp.int32)
                col = col + h
            incl = plsc.cumsum(col)
            offa = incl - col + pre + carry
            offa_v[pl.ds(z * 16, 16)] = offa
            offb_v[pl.ds(z * 16, 16)] = offa + hista_v[pl.ds(z * 16, 16)]
            return carry + jnp.sum(col)

    def rank_pass(shift):
        def digit(kk):
            if shift:
                return lax.shift_right_logical(kk, shift)
            return lax.bitwise_and(kk, K - 1)

        def rb(i, carry):
            ka = keys_v[pl.ds(i * 16, 16)]
            kb = keys_v[pl.ds(H + i * 16, 16)]
            da = digit(ka)
            db = digit(kb)
            ca, _ = plsc.scan_count(da)
            cb, _ = plsc.scan_count(db)
            basea = plsc.load_gather(offa_v, [da])
            baseb = plsc.load_gather(offb_v, [db])
            dst_store(i * 16, basea + ca - 1)
            dst_store(H + i * 16, baseb + cb - 1)
            plsc.addupdate_scatter(offa_v, [da], ones)
            plsc.addupdate_scatter(offb_v, [db], ones)
            return carry
        lax.fori_loop(0, VPH, rb, 0)

    def scatter_to(slot, buf_k, buf_v):
        copies = []
        for row in range(EP // 128):
            copies.append(pltpu.async_copy(
                keys_v.at[pl.ds(row * 128, 128)],
                buf_k.at[dst2_v.at[row]], sem))
            copies.append(pltpu.async_copy(
                val_v.at[pl.ds(slot * EP + row * 128, 128)],
                buf_v.at[dst2_v.at[row]], sem))
        for cp in copies:
            cp.wait()

    def in_base(j):
        return (c * BPC + j) * M + t * E

    def fire_prefetch(j):
        base = in_base(j)
        so = lax.bitwise_and(j, 1) * E
        sv = lax.bitwise_and(j, 1) * EP
        pltpu.async_copy(e0_hbm.at[pl.ds(base, E)],
                         e0_v.at[pl.ds(so, E)], sem_in)
        pltpu.async_copy(e1_hbm.at[pl.ds(base, E)],
                         e1_v.at[pl.ds(so, E)], sem_in)
        pltpu.async_copy(ev_hbm.at[pl.ds(base, E)],
                         val_v.at[pl.ds(sv, E)], sem_in)

    def drain_prefetch():
        pltpu.make_async_copy(
            e0_hbm.at[pl.ds(0, E)], e0_v.at[pl.ds(0, E)], sem_in).wait()
        pltpu.make_async_copy(
            e1_hbm.at[pl.ds(0, E)], e1_v.at[pl.ds(0, E)], sem_in).wait()
        pltpu.make_async_copy(
            ev_hbm.at[pl.ds(0, E)], val_v.at[pl.ds(0, E)], sem_in).wait()

    def drain_out():
        pltpu.make_async_copy(
            oc0s_v, oc0_hbm.at[pl.ds(0, E)], sem_out).wait()
        pltpu.make_async_copy(
            oc1s_v, oc1_hbm.at[pl.ds(0, E)], sem_out).wait()
        pltpu.make_async_copy(
            ovs_v, oval_hbm.at[pl.ds(0, E)], sem_out).wait()

    fire_prefetch(0)

    def batch(j, carry):
        b = c * BPC + j
        base = in_base(j)
        slot = lax.bitwise_and(j, 1)
        so = slot * E
        sv = slot * EP
        drain_prefetch()

        zero_hists()

        @plsc.parallel_loop(0, VPH, unroll=2)
        def _(i):
            e0a = e0_v[pl.ds(so + i * 16, 16)]
            e1a = e1_v[pl.ds(so + i * 16, 16)]
            e0b = e0_v[pl.ds(so + H + i * 16, 16)]
            e1b = e1_v[pl.ds(so + H + i * 16, 16)]
            keys_v[pl.ds(i * 16, 16)] = e0a * K + e1a
            keys_v[pl.ds(H + i * 16, 16)] = e0b * K + e1b
            plsc.addupdate_scatter(hista_v, [e1a], ones)
            plsc.addupdate_scatter(histb_v, [e1b], ones)
        fire_prefetch(jnp.minimum(j + 1, BPC - 1))
        merge_hists()
        pltpu.sync_copy(hist_v, grid_sh.at[pl.ds(t * K, K)])
        plsc.subcore_barrier()
        scan_offsets()
        rank_pass(0)
        scatter_to(slot, bufa_k, bufa_v)
        plsc.subcore_barrier()

        cpk = pltpu.async_copy(bufa_k.at[pl.ds(t * E, E)],
                               keys_v.at[pl.ds(0, E)], sem)
        cpv = pltpu.async_copy(bufa_v.at[pl.ds(t * E, E)],
                               val_v.at[pl.ds(sv, E)], sem)
        cpk.wait()
        cpv.wait()
        zero_hists()

        @plsc.parallel_loop(0, VPH, unroll=2)
        def _(i):
            ka = keys_v[pl.ds(i * 16, 16)]
            kb = keys_v[pl.ds(H + i * 16, 16)]
            plsc.addupdate_scatter(
                hista_v, [lax.shift_right_logical(ka, 10)], ones)
            plsc.addupdate_scatter(
                histb_v, [lax.shift_right_logical(kb, 10)], ones)
        merge_hists()
        pltpu.sync_copy(hist_v, grid_sh.at[pl.ds(t * K, K)])
        plsc.subcore_barrier()
        scan_offsets()
        rank_pass(10)
        scatter_to(slot, bufb_k, bufb_v)
        plsc.subcore_barrier()

        cpk2 = pltpu.async_copy(bufb_k.at[pl.ds(t * E, E)],
                                keys_v.at[pl.ds(0, E)], sem)
        cpv2 = pltpu.async_copy(bufb_v.at[pl.ds(t * E, E)],
                                val_v.at[pl.ds(sv, E)], sem)
        cpk2.wait()
        cpv2.wait()

        @pl.when(j > 0)
        def _():
            drain_out()

        base_node = b * N

        @plsc.parallel_loop(0, 2 * VPH, unroll=4)
        def _(i):
            kk = keys_v[pl.ds(i * 16, 16)]
            oc0s_v[pl.ds(i * 16, 16)] = \
                lax.shift_right_logical(kk, 10) + base_node
            oc1s_v[pl.ds(i * 16, 16)] = \
                lax.bitwise_and(kk, K - 1) + base_node
            ovs_v[pl.ds(i * 16, 16)] = val_v[pl.ds(sv + i * 16, 16)]
        pltpu.async_copy(oc0s_v, oc0_hbm.at[pl.ds(base, E)], sem_out)
        pltpu.async_copy(oc1s_v, oc1_hbm.at[pl.ds(base, E)], sem_out)
        pltpu.async_copy(ovs_v, oval_hbm.at[pl.ds(base, E)], sem_out)
        return carry

    lax.fori_loop(0, BPC, batch, 0)
    drain_prefetch()
    drain_out()


def kernel(nodes, edges, edge_index):
    b, n, f = nodes.shape
    m = edge_index.shape[1]
    e = m // T
    ep = ((e + 127) // 128) * 128
    e0f = edge_index[:, :, 0].reshape(b * m)
    e1f = edge_index[:, :, 1].reshape(b * m)
    ev = edges.reshape(b * m)
    mesh = plsc.VectorSubcoreMesh(core_axis_name="c", subcore_axis_name="s")
    fn = pl.kernel(
        functools.partial(_radix_body, (b, n, m)),
        out_type=(jax.ShapeDtypeStruct((b * m,), jnp.int32),
                  jax.ShapeDtypeStruct((b * m,), jnp.int32),
                  jax.ShapeDtypeStruct((b * m,), jnp.float32)),
        mesh=mesh,
        compiler_params=pltpu.CompilerParams(needs_layout_passes=False),
        scratch_types=[
            pltpu.VMEM((2 * e,), jnp.int32),
            pltpu.VMEM((2 * e,), jnp.int32),
            pltpu.VMEM((2 * ep,), jnp.float32),
            pltpu.VMEM((ep,), jnp.int32),
            pltpu.VMEM((ep // 128, 128), jnp.int32),
            pltpu.VMEM((K,), jnp.int32),
            pltpu.VMEM((K,), jnp.int32),
            pltpu.VMEM((K,), jnp.int32),
            pltpu.VMEM((K,), jnp.int32),
            pltpu.VMEM((K,), jnp.int32),
            pltpu.VMEM((T * K,), jnp.int32),
            pltpu.VMEM((e,), jnp.int32),
            pltpu.VMEM((e,), jnp.int32),
            pltpu.VMEM((e,), jnp.float32),
            pltpu.VMEM_SHARED((T * K,), jnp.int32),
            pltpu.VMEM_SHARED((m + 128,), jnp.int32),
            pltpu.VMEM_SHARED((m + 128,), jnp.float32),
            pltpu.VMEM_SHARED((m + 128,), jnp.int32),
            pltpu.VMEM_SHARED((m + 128,), jnp.float32),
            pltpu.SemaphoreType.DMA,
            pltpu.SemaphoreType.DMA,
            pltpu.SemaphoreType.DMA,
        ],
    )
    oc0, oc1, oval = fn(e0f, e1f, ev)
    indexlist = jnp.stack([oc0, oc1], axis=1).astype(jnp.int64)
    dense_shape = jnp.array([b * n, b * n], dtype=jnp.int64)
    return indexlist, oval, dense_shape
